# baseline (speedup 1.0000x reference)
import functools

import jax
import jax.numpy as jnp
from jax import lax
from jax.experimental import pallas as pl
from jax.experimental.pallas import tpu as pltpu
from jax.experimental.pallas import tpu_sc as plsc

BATCH = 4096
EMBED_DIM = 64


def _make_sc_gather(B, V, D):
    info = plsc.get_sparse_core_info()
    NC, NS = info.num_cores, info.num_subcores
    NW = NC * NS
    rows_per_w = D // NW
    mesh = plsc.VectorSubcoreMesh(core_axis_name="c", subcore_axis_name="s")

    @functools.partial(
        pl.kernel,
        mesh=mesh,
        out_type=[
            jax.ShapeDtypeStruct((D, B), jnp.float32),
            jax.ShapeDtypeStruct((D, B), jnp.float32),
        ],
        scratch_types=[
            pltpu.VMEM((B,), jnp.int32),
            pltpu.VMEM((B,), jnp.int32),
            pltpu.VMEM((V,), jnp.float32),
            pltpu.VMEM((4, B), jnp.float32),
        ],
        compiler_params=pltpu.CompilerParams(needs_layout_passes=False),
    )
    def gather_k(uid_hbm, mid_hbm, ut_hbm, mt_hbm, uout_hbm, mout_hbm,
                 uidx_v, midx_v, row_v, orows_v):
        wid = lax.axis_index("s") * NC + lax.axis_index("c")
        pltpu.sync_copy(uid_hbm, uidx_v)
        pltpu.sync_copy(mid_hbm, midx_v)

        def gather_row(idx_v, slot):
            def body(g, _):
                for j in range(8):
                    iv = idx_v[pl.ds(g * 128 + j * 16, 16)]
                    orows_v[slot, pl.ds(g * 128 + j * 16, 16)] = (
                        plsc.load_gather(row_v, [iv]))
                return 0
            lax.fori_loop(0, B // 128, body, 0)

        tabs = ((uidx_v, ut_hbm, uout_hbm), (midx_v, mt_hbm, mout_hbm))
        for t, (idx_v, t_hbm, _) in enumerate(tabs):
            for k in range(rows_per_w):
                r = wid + NW * k
                pltpu.sync_copy(t_hbm.at[r], row_v)
                gather_row(idx_v, t * rows_per_w + k)
        for t, (_, _, o_hbm) in enumerate(tabs):
            for k in range(rows_per_w):
                r = wid + NW * k
                pltpu.sync_copy(orows_v.at[t * rows_per_w + k], o_hbm.at[r])

    return gather_k


def _mlp_t_body(u_ref, m_ref, w1a_ref, w1b_ref, b1_ref, w2_ref, b2_ref,
                w3_ref, b3_ref, o_ref):
    cdim = (((0,), (0,)), ((), ()))
    h1 = lax.dot_general(w1a_ref[...], u_ref[...], cdim,
                         preferred_element_type=jnp.float32)
    h1 += lax.dot_general(w1b_ref[...], m_ref[...], cdim,
                          preferred_element_type=jnp.float32)
    h1 = jnp.maximum(h1 + b1_ref[...], 0.0)
    h2 = lax.dot_general(w2_ref[...], h1, cdim,
                         preferred_element_type=jnp.float32)
    h2 = jnp.maximum(h2 + b2_ref[...], 0.0)
    o_ref[...] = (
        lax.dot_general(h2, w3_ref[...], cdim,
                        preferred_element_type=jnp.float32)
        + b3_ref[...]
    )


def _make_mlp(B, D, BB):
    grid = (B // BB,)
    const = lambda i: (0, 0)
    return pl.pallas_call(
        _mlp_t_body,
        grid=grid,
        in_specs=[
            pl.BlockSpec((D, BB), lambda i: (0, i)),
            pl.BlockSpec((D, BB), lambda i: (0, i)),
            pl.BlockSpec((D, 256), const),
            pl.BlockSpec((D, 256), const),
            pl.BlockSpec((256, 1), const),
            pl.BlockSpec((256, 64), const),
            pl.BlockSpec((64, 1), const),
            pl.BlockSpec((64, 1), const),
            pl.BlockSpec((1, 1), const),
        ],
        out_specs=pl.BlockSpec((BB, 1), lambda i: (i, 0)),
        out_shape=jax.ShapeDtypeStruct((B, 1), jnp.float32),
    )


@jax.jit
def kernel(user_id, movie_id, user_table, movie_table, W1, b1, W2, b2, W3, b3):
    B = user_id.shape[0]
    V, D = user_table.shape
    gather_k = _make_sc_gather(B, V, D)
    u_t, m_t = gather_k(
        user_id.astype(jnp.int32), movie_id.astype(jnp.int32),
        user_table.T, movie_table.T)
    mlp = _make_mlp(B, D, 2048)
    return mlp(
        u_t, m_t,
        W1[:D], W1[D:],
        b1.reshape(256, 1),
        W2, b2.reshape(64, 1),
        W3, b3.reshape(1, 1),
    )


# speedup vs baseline: 1.0457x; 1.0457x over previous
import functools

import jax
import jax.numpy as jnp
from jax import lax
from jax.experimental import pallas as pl
from jax.experimental.pallas import tpu as pltpu
from jax.experimental.pallas import tpu_sc as plsc

BATCH = 4096
EMBED_DIM = 64


def _make_sc_gather(B, V, D):
    info = plsc.get_sparse_core_info()
    NC, NS = info.num_cores, info.num_subcores
    NW = NC * NS
    rows_per_w = D // NW
    mesh = plsc.VectorSubcoreMesh(core_axis_name="c", subcore_axis_name="s")

    @functools.partial(
        pl.kernel,
        mesh=mesh,
        out_type=[
            jax.ShapeDtypeStruct((D, B), jnp.float32),
            jax.ShapeDtypeStruct((D, B), jnp.float32),
        ],
        scratch_types=[
            pltpu.VMEM((B,), jnp.int32),
            pltpu.VMEM((B,), jnp.int32),
            pltpu.VMEM((50080,), jnp.float32),
            pltpu.VMEM((50080,), jnp.float32),
            pltpu.VMEM((4, B), jnp.float32),
            pltpu.SemaphoreType.DMA,
        ],
        compiler_params=pltpu.CompilerParams(needs_layout_passes=False),
    )
    def gather_k(uid_hbm, mid_hbm, ut_hbm, mt_hbm, uout_hbm, mout_hbm,
                 uidx_v, midx_v, buf_a, buf_b, orows_v, sem):
        H0 = 49920
        wid = lax.axis_index("s") * NC + lax.axis_index("c")
        pltpu.sync_copy(uid_hbm, uidx_v)
        pltpu.sync_copy(mid_hbm, midx_v)

        tabs = ((uidx_v, ut_hbm, uout_hbm), (midx_v, mt_hbm, mout_hbm))
        tasks = []
        for t, (idx_v, t_hbm, _) in enumerate(tabs):
            for k in range(rows_per_w):
                for h in range(2):
                    tasks.append((idx_v, t_hbm, t * rows_per_w + k, k, h))
        bufs = (buf_a, buf_b)
        descs = [None] * len(tasks)

        def start(i):
            _, t_hbm, _, k, h = tasks[i]
            r = wid + NW * k
            off, ln = (0, H0) if h == 0 else (H0, V - H0)
            descs[i] = pltpu.async_copy(
                t_hbm.at[r, pl.ds(off, ln)],
                bufs[i % 2].at[pl.ds(0, ln)], sem)

        def gather_half(i):
            idx_v, _, slot, _, h = tasks[i]
            buf = bufs[i % 2]

            def body(g, _):
                for j in range(8):
                    sl = pl.ds(g * 128 + j * 16, 16)
                    iv = idx_v[sl]
                    if h == 0:
                        got = plsc.load_gather(buf, [jnp.minimum(iv, H0 - 1)])
                        orows_v[slot, sl] = jnp.where(iv < H0, got, 0.0)
                    else:
                        got = plsc.load_gather(
                            buf, [jnp.maximum(iv - H0, 0)])
                        orows_v[slot, sl] = jnp.where(
                            iv >= H0, got, orows_v[slot, sl])
                return 0
            lax.fori_loop(0, B // 128, body, 0)

        start(0)
        for i in range(len(tasks)):
            if i + 1 < len(tasks):
                start(i + 1)
            descs[i].wait()
            gather_half(i)
        for t, (_, _, o_hbm) in enumerate(tabs):
            for k in range(rows_per_w):
                r = wid + NW * k
                pltpu.sync_copy(orows_v.at[t * rows_per_w + k], o_hbm.at[r])

    return gather_k


def _mlp_t_body(u_ref, m_ref, w1a_ref, w1b_ref, b1_ref, w2_ref, b2_ref,
                w3_ref, b3_ref, o_ref):
    cdim = (((0,), (0,)), ((), ()))
    h1 = lax.dot_general(w1a_ref[...], u_ref[...], cdim,
                         preferred_element_type=jnp.float32)
    h1 += lax.dot_general(w1b_ref[...], m_ref[...], cdim,
                          preferred_element_type=jnp.float32)
    h1 = jnp.maximum(h1 + b1_ref[...], 0.0)
    h2 = lax.dot_general(w2_ref[...], h1, cdim,
                         preferred_element_type=jnp.float32)
    h2 = jnp.maximum(h2 + b2_ref[...], 0.0)
    o_ref[...] = (
        lax.dot_general(w3_ref[...], h2, cdim,
                        preferred_element_type=jnp.float32)
        + b3_ref[...]
    )


def _make_mlp(B, D, BB):
    grid = (B // BB,)
    const = lambda i: (0, 0)
    return pl.pallas_call(
        _mlp_t_body,
        grid=grid,
        in_specs=[
            pl.BlockSpec((D, BB), lambda i: (0, i)),
            pl.BlockSpec((D, BB), lambda i: (0, i)),
            pl.BlockSpec((D, 256), const),
            pl.BlockSpec((D, 256), const),
            pl.BlockSpec((256, 1), const),
            pl.BlockSpec((256, 64), const),
            pl.BlockSpec((64, 1), const),
            pl.BlockSpec((64, 1), const),
            pl.BlockSpec((1, 1), const),
        ],
        out_specs=pl.BlockSpec((1, BB), lambda i: (0, i)),
        out_shape=jax.ShapeDtypeStruct((1, B), jnp.float32),
    )


@jax.jit
def kernel(user_id, movie_id, user_table, movie_table, W1, b1, W2, b2, W3, b3):
    B = user_id.shape[0]
    V, D = user_table.shape
    gather_k = _make_sc_gather(B, V, D)
    u_t, m_t = gather_k(
        user_id.astype(jnp.int32), movie_id.astype(jnp.int32),
        user_table.T, movie_table.T)
    mlp = _make_mlp(B, D, 1024)
    out_t = mlp(
        u_t, m_t,
        W1[:D], W1[D:],
        b1.reshape(256, 1),
        W2, b2.reshape(64, 1),
        W3, b3.reshape(1, 1),
    )
    return out_t.reshape(B, 1)
